# Initial kernel scaffold; baseline (speedup 1.0000x reference)
import jax
import jax.numpy as jnp
from jax.experimental import pallas as pl


def kernel(X, edge_index, W1, b1, W2, b2):
    raise NotImplementedError("write your pallas kernel here")


# speedup vs baseline: 14.6814x; 14.6814x over previous
import functools

import jax
import jax.numpy as jnp
from jax import lax
from jax.experimental import pallas as pl
from jax.experimental.pallas import tpu as pltpu
from jax.experimental.pallas import tpu_sc as plsc

N = 10000
NPAD = 10240
E = 320000
D = 128
NC = 2
NS = 16
NW = NC * NS
EPW = E // NW
B = 80
NB = EPW // B
ROWS_PT = NPAD // NS
DROWS_PT = NPAD // NS

_mesh = plsc.VectorSubcoreMesh(core_axis_name="c", subcore_axis_name="s")


@functools.partial(
    pl.kernel,
    mesh=_mesh,
    out_type=[
        jax.ShapeDtypeStruct((NPAD,), jnp.float32),
        jax.ShapeDtypeStruct((NPAD,), jnp.float32),
    ],
    scratch_types=[
        pltpu.VMEM_SHARED((NPAD,), jnp.float32),
        pltpu.VMEM((DROWS_PT,), jnp.float32),
        pltpu.VMEM((B,), jnp.float32),
        pltpu.VMEM((B,), jnp.int32),
    ],
)
def _deg_kernel(dst_hbm, out0, out1, acc, initv, val, didx):
    cid = lax.axis_index("c")
    sid = lax.axis_index("s")
    wid = sid * NC + cid

    fill = jnp.where(cid == 0, 1.0, 0.0).astype(jnp.float32)
    filled = jnp.broadcast_to(fill, (16,))

    def _fill_init(i, _):
        initv[pl.ds(pl.multiple_of(i * 16, 16), 16)] = filled
        return 0

    lax.fori_loop(0, DROWS_PT // 16, _fill_init, 0)

    def _fill_ones(i, _):
        val[pl.ds(pl.multiple_of(i * 16, 16), 16)] = jnp.ones((16,), jnp.float32)
        return 0

    lax.fori_loop(0, B // 16, _fill_ones, 0)

    ioff = pl.multiple_of(sid * DROWS_PT, 8)
    pltpu.sync_copy(initv, acc.at[pl.ds(ioff, DROWS_PT)])
    plsc.subcore_barrier()

    ebase = pl.multiple_of(wid * EPW, 8)

    def _edge_body(j, _):
        off = pl.multiple_of(ebase + j * B, 8)
        pltpu.sync_copy(dst_hbm.at[pl.ds(off, B)], didx)
        pltpu.sync_copy(val, acc.at[didx], add=True)
        return 0

    lax.fori_loop(0, NB, _edge_body, 0)

    plsc.subcore_barrier()

    @pl.when(cid == 0)
    def _():
        pltpu.sync_copy(acc.at[pl.ds(ioff, DROWS_PT)], out0.at[pl.ds(ioff, DROWS_PT)])

    @pl.when(cid == 1)
    def _():
        pltpu.sync_copy(acc.at[pl.ds(ioff, DROWS_PT)], out1.at[pl.ds(ioff, DROWS_PT)])


@functools.partial(
    pl.kernel,
    mesh=_mesh,
    out_type=[
        jax.ShapeDtypeStruct((NPAD, D), jnp.float32),
        jax.ShapeDtypeStruct((NPAD, D), jnp.float32),
    ],
    scratch_types=[
        pltpu.VMEM_SHARED((NPAD, D), jnp.float32),
        pltpu.VMEM((B, D), jnp.float32),
        pltpu.VMEM((B,), jnp.int32),
        pltpu.VMEM((B,), jnp.int32),
        pltpu.SemaphoreType.DMA,
    ],
)
def _scatter_kernel(x_hbm, zeros_hbm, src_hbm, dst_hbm, out0, out1,
                    acc, rows, sidx, didx, gsem):
    cid = lax.axis_index("c")
    sid = lax.axis_index("s")
    wid = sid * NC + cid

    roff = pl.multiple_of(sid * ROWS_PT, 8)
    @pl.when(cid == 0)
    def _():
        pltpu.sync_copy(x_hbm.at[pl.ds(roff, ROWS_PT)], acc.at[pl.ds(roff, ROWS_PT)])

    @pl.when(cid == 1)
    def _():
        pltpu.sync_copy(zeros_hbm.at[pl.ds(roff, ROWS_PT)], acc.at[pl.ds(roff, ROWS_PT)])

    plsc.subcore_barrier()

    ebase = pl.multiple_of(wid * EPW, 8)

    def _edge_body(j, _):
        off = pl.multiple_of(ebase + j * B, 8)
        pltpu.sync_copy(src_hbm.at[pl.ds(off, B)], sidx)
        pltpu.sync_copy(dst_hbm.at[pl.ds(off, B)], didx)
        pltpu.async_copy(x_hbm.at[sidx], rows, gsem).wait()
        pltpu.sync_copy(rows, acc.at[didx], add=True)
        return 0

    lax.fori_loop(0, NB, _edge_body, 0)

    plsc.subcore_barrier()

    @pl.when(cid == 0)
    def _():
        pltpu.sync_copy(acc.at[pl.ds(roff, ROWS_PT)], out0.at[pl.ds(roff, ROWS_PT)])

    @pl.when(cid == 1)
    def _():
        pltpu.sync_copy(acc.at[pl.ds(roff, ROWS_PT)], out1.at[pl.ds(roff, ROWS_PT)])


def _scale_body(x_ref, d0_ref, d1_ref, o_ref):
    dis = lax.rsqrt(d0_ref[...] + d1_ref[...])
    o_ref[...] = x_ref[...] * dis


def _scale_x(x, d0, d1):
    return pl.pallas_call(
        _scale_body,
        out_shape=jax.ShapeDtypeStruct((NPAD, D), jnp.float32),
    )(x, d0, d1)


_RB = 2048


def _mid_body(r0_ref, r1_ref, d0_ref, d1_ref, w1_ref, b1_ref, w2_ref, o_ref):
    dis = lax.rsqrt(d0_ref[...] + d1_ref[...])
    y = (r0_ref[...] + r1_ref[...]) * dis
    h = jnp.dot(y, w1_ref[...], preferred_element_type=jnp.float32) + b1_ref[...]
    t = jnp.dot(h, w2_ref[...], preferred_element_type=jnp.float32)
    o_ref[...] = t * dis


def _mid(r0, r1, d0, d1, W1, b1, W2):
    grid = (NPAD // _RB,)
    row_spec = pl.BlockSpec((_RB, D), lambda i: (i, 0))
    deg_spec = pl.BlockSpec((_RB, 1), lambda i: (i, 0))
    return pl.pallas_call(
        _mid_body,
        grid=grid,
        in_specs=[
            row_spec,
            row_spec,
            deg_spec,
            deg_spec,
            pl.BlockSpec((D, 256), lambda i: (0, 0)),
            pl.BlockSpec((256,), lambda i: (0,)),
            pl.BlockSpec((256, D), lambda i: (0, 0)),
        ],
        out_specs=row_spec,
        out_shape=jax.ShapeDtypeStruct((NPAD, D), jnp.float32),
    )(r0, r1, d0, d1, W1, b1, W2)


def _final_body(q0_ref, q1_ref, d0_ref, d1_ref, b2_ref, o_ref):
    dis = lax.rsqrt(d0_ref[...] + d1_ref[...])
    o_ref[...] = jnp.maximum((q0_ref[...] + q1_ref[...]) * dis + b2_ref[...], 0.0)


def _final(q0, q1, d0, d1, b2):
    grid = (NPAD // _RB,)
    row_spec = pl.BlockSpec((_RB, D), lambda i: (i, 0))
    deg_spec = pl.BlockSpec((_RB, 1), lambda i: (i, 0))
    return pl.pallas_call(
        _final_body,
        grid=grid,
        in_specs=[
            row_spec,
            row_spec,
            deg_spec,
            deg_spec,
            pl.BlockSpec((D,), lambda i: (0,)),
        ],
        out_specs=row_spec,
        out_shape=jax.ShapeDtypeStruct((NPAD, D), jnp.float32),
    )(q0, q1, d0, d1, b2)


def kernel(X, edge_index, W1, b1, W2, b2):
    src = edge_index[0].astype(jnp.int32)
    dst = edge_index[1].astype(jnp.int32)
    xpad = jnp.pad(X, ((0, NPAD - N), (0, 0)))
    zeros = jnp.zeros((NPAD, D), jnp.float32)

    dp0, dp1 = _deg_kernel(dst)
    d0 = dp0.reshape(NPAD, 1)
    d1 = dp1.reshape(NPAD, 1)

    x0 = _scale_x(xpad, d0, d1)
    r0, r1 = _scatter_kernel(x0, zeros, src, dst)
    t0 = _mid(r0, r1, d0, d1, W1, b1, W2)
    q0, q1 = _scatter_kernel(t0, zeros, src, dst)
    return _final(q0, q1, d0, d1, b2)[:N]


# speedup vs baseline: 27.4024x; 1.8665x over previous
import functools

import jax
import jax.numpy as jnp
from jax import lax
from jax.experimental import pallas as pl
from jax.experimental.pallas import tpu as pltpu
from jax.experimental.pallas import tpu_sc as plsc

N = 10000
NPAD = 10240
E = 320000
D = 128
NC = 2
NS = 16
NW = NC * NS
EPW = E // NW
B = 80
NB = EPW // B
ROWS_PT = NPAD // NS
DROWS_PT = NPAD // NS

_mesh = plsc.VectorSubcoreMesh(core_axis_name="c", subcore_axis_name="s")


NQ = 2
NQI = NB // NQ


def _stage_idx(dst_buf, src_buf, off):
    for k in range(B // 16):
        o = pl.multiple_of(off + 16 * k, 8)
        dst_buf[pl.ds(pl.multiple_of(16 * k, 8), 16)] = src_buf[pl.ds(o, 16)]


@functools.partial(
    pl.kernel,
    mesh=_mesh,
    out_type=[
        jax.ShapeDtypeStruct((NPAD,), jnp.float32),
        jax.ShapeDtypeStruct((NPAD,), jnp.float32),
    ],
    scratch_types=[
        pltpu.VMEM_SHARED((NPAD,), jnp.float32),
        pltpu.VMEM((DROWS_PT,), jnp.float32),
        pltpu.VMEM((B,), jnp.float32),
        pltpu.VMEM((EPW,), jnp.int32),
    ]
    + [pltpu.VMEM((B,), jnp.int32) for _ in range(NQ)]
    + [pltpu.SemaphoreType.DMA for _ in range(NQ)],
)
def _deg_kernel(dst_hbm, out0, out1, acc, initv, val, didx_all, *rest):
    didx_b = rest[:NQ]
    ssem = rest[NQ:]
    cid = lax.axis_index("c")
    sid = lax.axis_index("s")
    wid = sid * NC + cid

    fill = jnp.where(cid == 0, 1.0, 0.0).astype(jnp.float32)
    filled = jnp.broadcast_to(fill, (16,))

    def _fill_init(i, _):
        initv[pl.ds(pl.multiple_of(i * 16, 16), 16)] = filled
        return 0

    lax.fori_loop(0, DROWS_PT // 16, _fill_init, 0)

    def _fill_ones(i, _):
        val[pl.ds(pl.multiple_of(i * 16, 16), 16)] = jnp.ones((16,), jnp.float32)
        return 0

    lax.fori_loop(0, B // 16, _fill_ones, 0)

    ebase = pl.multiple_of(wid * EPW, 8)
    pltpu.sync_copy(dst_hbm.at[pl.ds(ebase, EPW)], didx_all)

    ioff = pl.multiple_of(sid * DROWS_PT, 8)
    pltpu.sync_copy(initv, acc.at[pl.ds(ioff, DROWS_PT)])
    plsc.subcore_barrier()

    def _quad(i, _):
        for k in range(NQ):
            _stage_idx(didx_b[k], didx_all, (i * NQ + k) * B)
        descs = [
            pltpu.async_copy(val, acc.at[didx_b[k]], ssem[k], add=True)
            for k in range(NQ)
        ]
        for d in descs:
            d.wait()
        return 0

    lax.fori_loop(0, NQI, _quad, 0)
    _stage_idx(didx_b[0], didx_all, (NB - 1) * B)
    pltpu.sync_copy(val, acc.at[didx_b[0]], add=True)

    plsc.subcore_barrier()

    @pl.when(cid == 0)
    def _():
        pltpu.sync_copy(acc.at[pl.ds(ioff, DROWS_PT)], out0.at[pl.ds(ioff, DROWS_PT)])

    @pl.when(cid == 1)
    def _():
        pltpu.sync_copy(acc.at[pl.ds(ioff, DROWS_PT)], out1.at[pl.ds(ioff, DROWS_PT)])


@functools.partial(
    pl.kernel,
    mesh=_mesh,
    out_type=[
        jax.ShapeDtypeStruct((NPAD, D), jnp.float32),
        jax.ShapeDtypeStruct((NPAD, D), jnp.float32),
    ],
    scratch_types=[
        pltpu.VMEM_SHARED((NPAD, D), jnp.float32),
        pltpu.VMEM((EPW,), jnp.int32),
        pltpu.VMEM((EPW,), jnp.int32),
    ]
    + [pltpu.VMEM((B, D), jnp.float32) for _ in range(NQ)]
    + [pltpu.VMEM((B,), jnp.int32) for _ in range(NQ)]
    + [pltpu.VMEM((B,), jnp.int32) for _ in range(NQ)]
    + [pltpu.SemaphoreType.DMA for _ in range(2 * NQ)],
)
def _scatter_kernel(x_hbm, zeros_hbm, src_hbm, dst_hbm, out0, out1,
                    acc, sidx_all, didx_all, *rest):
    rows = rest[:NQ]
    sidx_b = rest[NQ:2 * NQ]
    didx_b = rest[2 * NQ:3 * NQ]
    gsem = rest[3 * NQ:4 * NQ]
    ssem = rest[4 * NQ:5 * NQ]
    cid = lax.axis_index("c")
    sid = lax.axis_index("s")
    wid = sid * NC + cid

    ebase = pl.multiple_of(wid * EPW, 8)
    pltpu.sync_copy(src_hbm.at[pl.ds(ebase, EPW)], sidx_all)
    pltpu.sync_copy(dst_hbm.at[pl.ds(ebase, EPW)], didx_all)

    roff = pl.multiple_of(sid * ROWS_PT, 8)
    @pl.when(cid == 0)
    def _():
        pltpu.sync_copy(x_hbm.at[pl.ds(roff, ROWS_PT)], acc.at[pl.ds(roff, ROWS_PT)])

    @pl.when(cid == 1)
    def _():
        pltpu.sync_copy(zeros_hbm.at[pl.ds(roff, ROWS_PT)], acc.at[pl.ds(roff, ROWS_PT)])

    plsc.subcore_barrier()

    def _quad(i, _):
        gdescs = []
        for k in range(NQ):
            _stage_idx(sidx_b[k], sidx_all, (i * NQ + k) * B)
            gdescs.append(
                pltpu.async_copy(x_hbm.at[sidx_b[k]], rows[k], gsem[k]))
        for k in range(NQ):
            _stage_idx(didx_b[k], didx_all, (i * NQ + k) * B)
        sdescs = []
        for k in range(NQ):
            gdescs[k].wait()
            sdescs.append(
                pltpu.async_copy(rows[k], acc.at[didx_b[k]], ssem[k], add=True))
        for d in sdescs:
            d.wait()
        return 0

    lax.fori_loop(0, NQI, _quad, 0)

    _stage_idx(sidx_b[0], sidx_all, (NB - 1) * B)
    _stage_idx(didx_b[0], didx_all, (NB - 1) * B)
    pltpu.async_copy(x_hbm.at[sidx_b[0]], rows[0], gsem[0]).wait()
    pltpu.sync_copy(rows[0], acc.at[didx_b[0]], add=True)

    plsc.subcore_barrier()

    @pl.when(cid == 0)
    def _():
        pltpu.sync_copy(acc.at[pl.ds(roff, ROWS_PT)], out0.at[pl.ds(roff, ROWS_PT)])

    @pl.when(cid == 1)
    def _():
        pltpu.sync_copy(acc.at[pl.ds(roff, ROWS_PT)], out1.at[pl.ds(roff, ROWS_PT)])


def _scale_body(x_ref, d0_ref, d1_ref, o_ref):
    dis = lax.rsqrt(d0_ref[...] + d1_ref[...])
    o_ref[...] = x_ref[...] * dis


def _scale_x(x, d0, d1):
    return pl.pallas_call(
        _scale_body,
        out_shape=jax.ShapeDtypeStruct((NPAD, D), jnp.float32),
    )(x, d0, d1)


_RB = 2048


def _mid_body(r0_ref, r1_ref, d0_ref, d1_ref, w1_ref, b1_ref, w2_ref, o_ref):
    dis = lax.rsqrt(d0_ref[...] + d1_ref[...])
    y = (r0_ref[...] + r1_ref[...]) * dis
    h = jnp.dot(y, w1_ref[...], preferred_element_type=jnp.float32) + b1_ref[...]
    t = jnp.dot(h, w2_ref[...], preferred_element_type=jnp.float32)
    o_ref[...] = t * dis


def _mid(r0, r1, d0, d1, W1, b1, W2):
    grid = (NPAD // _RB,)
    row_spec = pl.BlockSpec((_RB, D), lambda i: (i, 0))
    deg_spec = pl.BlockSpec((_RB, 1), lambda i: (i, 0))
    return pl.pallas_call(
        _mid_body,
        grid=grid,
        in_specs=[
            row_spec,
            row_spec,
            deg_spec,
            deg_spec,
            pl.BlockSpec((D, 256), lambda i: (0, 0)),
            pl.BlockSpec((256,), lambda i: (0,)),
            pl.BlockSpec((256, D), lambda i: (0, 0)),
        ],
        out_specs=row_spec,
        out_shape=jax.ShapeDtypeStruct((NPAD, D), jnp.float32),
    )(r0, r1, d0, d1, W1, b1, W2)


def _final_body(q0_ref, q1_ref, d0_ref, d1_ref, b2_ref, o_ref):
    dis = lax.rsqrt(d0_ref[...] + d1_ref[...])
    o_ref[...] = jnp.maximum((q0_ref[...] + q1_ref[...]) * dis + b2_ref[...], 0.0)


def _final(q0, q1, d0, d1, b2):
    grid = (NPAD // _RB,)
    row_spec = pl.BlockSpec((_RB, D), lambda i: (i, 0))
    deg_spec = pl.BlockSpec((_RB, 1), lambda i: (i, 0))
    return pl.pallas_call(
        _final_body,
        grid=grid,
        in_specs=[
            row_spec,
            row_spec,
            deg_spec,
            deg_spec,
            pl.BlockSpec((D,), lambda i: (0,)),
        ],
        out_specs=row_spec,
        out_shape=jax.ShapeDtypeStruct((NPAD, D), jnp.float32),
    )(q0, q1, d0, d1, b2)


def kernel(X, edge_index, W1, b1, W2, b2):
    src = edge_index[0].astype(jnp.int32)
    dst = edge_index[1].astype(jnp.int32)
    xpad = jnp.pad(X, ((0, NPAD - N), (0, 0)))
    zeros = jnp.zeros((NPAD, D), jnp.float32)

    dp0, dp1 = _deg_kernel(dst)
    d0 = dp0.reshape(NPAD, 1)
    d1 = dp1.reshape(NPAD, 1)

    x0 = _scale_x(xpad, d0, d1)
    r0, r1 = _scatter_kernel(x0, zeros, src, dst)
    t0 = _mid(r0, r1, d0, d1, W1, b1, W2)
    q0, q1 = _scatter_kernel(t0, zeros, src, dst)
    return _final(q0, q1, d0, d1, b2)[:N]


# speedup vs baseline: 29.7118x; 1.0843x over previous
import functools

import jax
import jax.numpy as jnp
from jax import lax
from jax.experimental import pallas as pl
from jax.experimental.pallas import tpu as pltpu
from jax.experimental.pallas import tpu_sc as plsc

N = 10000
NPAD = 10240
E = 320000
D = 128
NC = 2
NS = 16
NW = NC * NS
EPW = E // NW
B = 80
NB = EPW // B
ROWS_PT = NPAD // NS
DROWS_PT = NPAD // NS

_mesh = plsc.VectorSubcoreMesh(core_axis_name="c", subcore_axis_name="s")


NQ = 2
NQI = NB // NQ
QN = 4
QD = B
NQUAD = NB // QN


def _stage_idx(dst_buf, src_buf, off):
    for k in range(B // 16):
        o = pl.multiple_of(off + 16 * k, 8)
        dst_buf[pl.ds(pl.multiple_of(16 * k, 8), 16)] = src_buf[pl.ds(o, 16)]


@functools.partial(
    pl.kernel,
    mesh=_mesh,
    out_type=[
        jax.ShapeDtypeStruct((NPAD,), jnp.float32),
        jax.ShapeDtypeStruct((NPAD,), jnp.float32),
    ],
    scratch_types=[
        pltpu.VMEM_SHARED((NPAD,), jnp.float32),
        pltpu.VMEM((DROWS_PT,), jnp.float32),
        pltpu.VMEM((B,), jnp.float32),
        pltpu.VMEM((EPW,), jnp.int32),
    ]
    + [pltpu.VMEM((B,), jnp.int32) for _ in range(NQ)]
    + [pltpu.SemaphoreType.DMA for _ in range(NQ)],
)
def _deg_kernel(dst_hbm, out0, out1, acc, initv, val, didx_all, *rest):
    didx_b = rest[:NQ]
    ssem = rest[NQ:]
    cid = lax.axis_index("c")
    sid = lax.axis_index("s")
    wid = sid * NC + cid

    fill = jnp.where(cid == 0, 1.0, 0.0).astype(jnp.float32)
    filled = jnp.broadcast_to(fill, (16,))

    def _fill_init(i, _):
        initv[pl.ds(pl.multiple_of(i * 16, 16), 16)] = filled
        return 0

    lax.fori_loop(0, DROWS_PT // 16, _fill_init, 0)

    def _fill_ones(i, _):
        val[pl.ds(pl.multiple_of(i * 16, 16), 16)] = jnp.ones((16,), jnp.float32)
        return 0

    lax.fori_loop(0, B // 16, _fill_ones, 0)

    ebase = pl.multiple_of(wid * EPW, 8)
    pltpu.sync_copy(dst_hbm.at[pl.ds(ebase, EPW)], didx_all)

    ioff = pl.multiple_of(sid * DROWS_PT, 8)
    pltpu.sync_copy(initv, acc.at[pl.ds(ioff, DROWS_PT)])
    plsc.subcore_barrier()

    def _quad(i, _):
        for k in range(NQ):
            _stage_idx(didx_b[k], didx_all, (i * NQ + k) * B)
        descs = [
            pltpu.async_copy(val, acc.at[didx_b[k]], ssem[k], add=True)
            for k in range(NQ)
        ]
        for d in descs:
            d.wait()
        return 0

    lax.fori_loop(0, NQI, _quad, 0)
    _stage_idx(didx_b[0], didx_all, (NB - 1) * B)
    pltpu.sync_copy(val, acc.at[didx_b[0]], add=True)

    plsc.subcore_barrier()

    @pl.when(cid == 0)
    def _():
        pltpu.sync_copy(acc.at[pl.ds(ioff, DROWS_PT)], out0.at[pl.ds(ioff, DROWS_PT)])

    @pl.when(cid == 1)
    def _():
        pltpu.sync_copy(acc.at[pl.ds(ioff, DROWS_PT)], out1.at[pl.ds(ioff, DROWS_PT)])


@functools.partial(
    pl.kernel,
    mesh=_mesh,
    out_type=[
        jax.ShapeDtypeStruct((NPAD, D), jnp.float32),
        jax.ShapeDtypeStruct((NPAD, D), jnp.float32),
    ],
    scratch_types=[
        pltpu.VMEM_SHARED((NPAD, D), jnp.float32),
    ]
    + [pltpu.VMEM((QD, D), jnp.float32) for _ in range(QN)]
    + [pltpu.VMEM((B,), jnp.int32) for _ in range(2 * QN)]
    + [pltpu.VMEM((B,), jnp.int32) for _ in range(2 * QN)]
    + [pltpu.SemaphoreType.DMA for _ in range(2 + 2 * QN)],
)
def _scatter_kernel(x_hbm, zeros_hbm, src_hbm, dst_hbm, out0, out1,
                    acc, *rest):
    rows = rest[:QN]
    sidx = [rest[QN + QN * p:QN + QN * p + QN] for p in range(2)]
    didx = [rest[3 * QN + QN * p:3 * QN + QN * p + QN] for p in range(2)]
    isem = rest[5 * QN:5 * QN + 2]
    gsem = rest[5 * QN + 2:6 * QN + 2]
    ssem = rest[6 * QN + 2:7 * QN + 2]
    cid = lax.axis_index("c")
    sid = lax.axis_index("s")
    wid = sid * NC + cid

    ebase = pl.multiple_of(wid * EPW, 8)

    def _prefetch(i, p):
        for k in range(QN):
            off = pl.multiple_of(ebase + (i * QN + k) * B, 8)
            pltpu.async_copy(src_hbm.at[pl.ds(off, B)], sidx[p][k], isem[p])
            pltpu.async_copy(dst_hbm.at[pl.ds(off, B)], didx[p][k], isem[p])

    def _drain_idx(i, p):
        for k in range(QN):
            off = pl.multiple_of(ebase + (i * QN + k) * B, 8)
            pltpu.make_async_copy(src_hbm.at[pl.ds(off, B)], sidx[p][k], isem[p]).wait()
            pltpu.make_async_copy(dst_hbm.at[pl.ds(off, B)], didx[p][k], isem[p]).wait()

    _prefetch(0, 0)

    roff = pl.multiple_of(sid * ROWS_PT, 8)
    @pl.when(cid == 0)
    def _():
        pltpu.sync_copy(x_hbm.at[pl.ds(roff, ROWS_PT)], acc.at[pl.ds(roff, ROWS_PT)])

    @pl.when(cid == 1)
    def _():
        pltpu.sync_copy(zeros_hbm.at[pl.ds(roff, ROWS_PT)], acc.at[pl.ds(roff, ROWS_PT)])

    plsc.subcore_barrier()

    def _quad_body(i, p):
        @pl.when(i + 1 < NQUAD)
        def _():
            _prefetch(i + 1, 1 - p)

        _drain_idx(i, p)
        gdescs = [
            pltpu.async_copy(x_hbm.at[sidx[p][k]], rows[k], gsem[k])
            for k in range(QN)
        ]
        sdescs = []
        for k in range(QN):
            gdescs[k].wait()
            sdescs.append(
                pltpu.async_copy(rows[k], acc.at[didx[p][k]], ssem[k], add=True))
        for d in sdescs:
            d.wait()

    def _quad(i, _):
        @pl.when(i % 2 == 0)
        def _():
            _quad_body(i, 0)

        @pl.when(i % 2 == 1)
        def _():
            _quad_body(i, 1)

        return 0

    lax.fori_loop(0, NQUAD, _quad, 0)

    eoff = pl.multiple_of(ebase + (NB - 1) * B, 8)
    pltpu.sync_copy(src_hbm.at[pl.ds(eoff, B)], sidx[0][0])
    pltpu.sync_copy(dst_hbm.at[pl.ds(eoff, B)], didx[0][0])
    pltpu.async_copy(x_hbm.at[sidx[0][0]], rows[0], gsem[0]).wait()
    pltpu.sync_copy(rows[0], acc.at[didx[0][0]], add=True)

    plsc.subcore_barrier()

    @pl.when(cid == 0)
    def _():
        pltpu.sync_copy(acc.at[pl.ds(roff, ROWS_PT)], out0.at[pl.ds(roff, ROWS_PT)])

    @pl.when(cid == 1)
    def _():
        pltpu.sync_copy(acc.at[pl.ds(roff, ROWS_PT)], out1.at[pl.ds(roff, ROWS_PT)])


def _scale_body(x_ref, d0_ref, d1_ref, o_ref):
    dis = lax.rsqrt(d0_ref[...] + d1_ref[...])
    o_ref[...] = x_ref[...] * dis


def _scale_x(x, d0, d1):
    return pl.pallas_call(
        _scale_body,
        out_shape=jax.ShapeDtypeStruct((NPAD, D), jnp.float32),
    )(x, d0, d1)


_RB = 2048


def _mid_body(r0_ref, r1_ref, d0_ref, d1_ref, w1_ref, b1_ref, w2_ref, o_ref):
    dis = lax.rsqrt(d0_ref[...] + d1_ref[...])
    y = (r0_ref[...] + r1_ref[...]) * dis
    h = jnp.dot(y, w1_ref[...], preferred_element_type=jnp.float32) + b1_ref[...]
    t = jnp.dot(h, w2_ref[...], preferred_element_type=jnp.float32)
    o_ref[...] = t * dis


def _mid(r0, r1, d0, d1, W1, b1, W2):
    grid = (NPAD // _RB,)
    row_spec = pl.BlockSpec((_RB, D), lambda i: (i, 0))
    deg_spec = pl.BlockSpec((_RB, 1), lambda i: (i, 0))
    return pl.pallas_call(
        _mid_body,
        grid=grid,
        in_specs=[
            row_spec,
            row_spec,
            deg_spec,
            deg_spec,
            pl.BlockSpec((D, 256), lambda i: (0, 0)),
            pl.BlockSpec((256,), lambda i: (0,)),
            pl.BlockSpec((256, D), lambda i: (0, 0)),
        ],
        out_specs=row_spec,
        out_shape=jax.ShapeDtypeStruct((NPAD, D), jnp.float32),
    )(r0, r1, d0, d1, W1, b1, W2)


def _final_body(q0_ref, q1_ref, d0_ref, d1_ref, b2_ref, o_ref):
    dis = lax.rsqrt(d0_ref[...] + d1_ref[...])
    o_ref[...] = jnp.maximum((q0_ref[...] + q1_ref[...]) * dis + b2_ref[...], 0.0)


def _final(q0, q1, d0, d1, b2):
    grid = (NPAD // _RB,)
    row_spec = pl.BlockSpec((_RB, D), lambda i: (i, 0))
    deg_spec = pl.BlockSpec((_RB, 1), lambda i: (i, 0))
    return pl.pallas_call(
        _final_body,
        grid=grid,
        in_specs=[
            row_spec,
            row_spec,
            deg_spec,
            deg_spec,
            pl.BlockSpec((D,), lambda i: (0,)),
        ],
        out_specs=row_spec,
        out_shape=jax.ShapeDtypeStruct((NPAD, D), jnp.float32),
    )(q0, q1, d0, d1, b2)


def kernel(X, edge_index, W1, b1, W2, b2):
    src = edge_index[0].astype(jnp.int32)
    dst = edge_index[1].astype(jnp.int32)
    xpad = jnp.pad(X, ((0, NPAD - N), (0, 0)))
    zeros = jnp.zeros((NPAD, D), jnp.float32)

    dp0, dp1 = _deg_kernel(dst)
    d0 = dp0.reshape(NPAD, 1)
    d1 = dp1.reshape(NPAD, 1)

    x0 = _scale_x(xpad, d0, d1)
    r0, r1 = _scatter_kernel(x0, zeros, src, dst)
    t0 = _mid(r0, r1, d0, d1, W1, b1, W2)
    q0, q1 = _scatter_kernel(t0, zeros, src, dst)
    return _final(q0, q1, d0, d1, b2)[:N]


# speedup vs baseline: 37.1916x; 1.2517x over previous
import functools

import jax
import jax.numpy as jnp
from jax import lax
from jax.experimental import pallas as pl
from jax.experimental.pallas import tpu as pltpu
from jax.experimental.pallas import tpu_sc as plsc

N = 10000
NPAD = 10240
E = 320000
D = 128
NC = 2
NS = 16
NW = NC * NS
EPW = E // NW
B = 80
NB = EPW // B
ROWS_PT = NPAD // NS
DROWS_PT = NPAD // NS

_mesh = plsc.VectorSubcoreMesh(core_axis_name="c", subcore_axis_name="s")


NQ = 2
NQI = NB // NQ
QN = 2
NQUAD = NB // QN


def _stage_idx(dst_buf, src_buf, off):
    for k in range(B // 16):
        o = pl.multiple_of(off + 16 * k, 8)
        dst_buf[pl.ds(pl.multiple_of(16 * k, 8), 16)] = src_buf[pl.ds(o, 16)]


@functools.partial(
    pl.kernel,
    mesh=_mesh,
    out_type=[
        jax.ShapeDtypeStruct((NPAD,), jnp.float32),
        jax.ShapeDtypeStruct((NPAD,), jnp.float32),
    ],
    scratch_types=[
        pltpu.VMEM_SHARED((NPAD,), jnp.float32),
        pltpu.VMEM((DROWS_PT,), jnp.float32),
        pltpu.VMEM((B,), jnp.float32),
        pltpu.VMEM((EPW,), jnp.int32),
    ]
    + [pltpu.VMEM((B,), jnp.int32) for _ in range(NQ)]
    + [pltpu.SemaphoreType.DMA for _ in range(NQ)],
)
def _deg_kernel(dst_hbm, out0, out1, acc, initv, val, didx_all, *rest):
    didx_b = rest[:NQ]
    ssem = rest[NQ:]
    cid = lax.axis_index("c")
    sid = lax.axis_index("s")
    wid = sid * NC + cid

    fill = jnp.where(cid == 0, 1.0, 0.0).astype(jnp.float32)
    filled = jnp.broadcast_to(fill, (16,))

    def _fill_init(i, _):
        initv[pl.ds(pl.multiple_of(i * 16, 16), 16)] = filled
        return 0

    lax.fori_loop(0, DROWS_PT // 16, _fill_init, 0)

    def _fill_ones(i, _):
        val[pl.ds(pl.multiple_of(i * 16, 16), 16)] = jnp.ones((16,), jnp.float32)
        return 0

    lax.fori_loop(0, B // 16, _fill_ones, 0)

    ebase = pl.multiple_of(wid * EPW, 8)
    pltpu.sync_copy(dst_hbm.at[pl.ds(ebase, EPW)], didx_all)

    ioff = pl.multiple_of(sid * DROWS_PT, 8)
    pltpu.sync_copy(initv, acc.at[pl.ds(ioff, DROWS_PT)])
    plsc.subcore_barrier()

    def _quad(i, _):
        for k in range(NQ):
            _stage_idx(didx_b[k], didx_all, (i * NQ + k) * B)
        descs = [
            pltpu.async_copy(val, acc.at[didx_b[k]], ssem[k], add=True)
            for k in range(NQ)
        ]
        for d in descs:
            d.wait()
        return 0

    lax.fori_loop(0, NQI, _quad, 0)
    _stage_idx(didx_b[0], didx_all, (NB - 1) * B)
    pltpu.sync_copy(val, acc.at[didx_b[0]], add=True)

    plsc.subcore_barrier()

    @pl.when(cid == 0)
    def _():
        pltpu.sync_copy(acc.at[pl.ds(ioff, DROWS_PT)], out0.at[pl.ds(ioff, DROWS_PT)])

    @pl.when(cid == 1)
    def _():
        pltpu.sync_copy(acc.at[pl.ds(ioff, DROWS_PT)], out1.at[pl.ds(ioff, DROWS_PT)])


@functools.partial(
    pl.kernel,
    mesh=_mesh,
    out_type=[
        jax.ShapeDtypeStruct((NPAD, D), jnp.float32),
        jax.ShapeDtypeStruct((NPAD, D), jnp.float32),
    ],
    scratch_types=[
        pltpu.VMEM_SHARED((NPAD, D), jnp.float32),
    ]
    + [pltpu.VMEM((B, D), jnp.float32) for _ in range(2 * QN)]
    + [pltpu.VMEM((B,), jnp.int32) for _ in range(2 * QN)]
    + [pltpu.VMEM((B,), jnp.int32) for _ in range(2 * QN)]
    + [pltpu.SemaphoreType.DMA for _ in range(8)],
)
def _scatter_kernel(x_hbm, zeros_hbm, src_hbm, dst_hbm, out0, out1,
                    acc, *rest):
    rows = [rest[QN * p:QN * p + QN] for p in range(2)]
    sidx = [rest[2 * QN + QN * p:3 * QN + QN * p] for p in range(2)]
    didx = [rest[4 * QN + QN * p:5 * QN + QN * p] for p in range(2)]
    is_sem = rest[6 * QN:6 * QN + 2]
    id_sem = rest[6 * QN + 2:6 * QN + 4]
    gsem = rest[6 * QN + 4:6 * QN + 6]
    ssem = rest[6 * QN + 6:6 * QN + 8]
    cid = lax.axis_index("c")
    sid = lax.axis_index("s")
    wid = sid * NC + cid

    ebase = pl.multiple_of(wid * EPW, 8)

    def _eoff(i, k):
        return pl.multiple_of(ebase + (i * QN + k) * B, 8)

    def _fire_sidx(i, p):
        for k in range(QN):
            pltpu.async_copy(src_hbm.at[pl.ds(_eoff(i, k), B)], sidx[p][k], is_sem[p])

    def _drain_sidx(i, p):
        for k in range(QN):
            pltpu.make_async_copy(
                src_hbm.at[pl.ds(_eoff(i, k), B)], sidx[p][k], is_sem[p]).wait()

    def _fire_didx(i, p):
        for k in range(QN):
            pltpu.async_copy(dst_hbm.at[pl.ds(_eoff(i, k), B)], didx[p][k], id_sem[p])

    def _drain_didx(i, p):
        for k in range(QN):
            pltpu.make_async_copy(
                dst_hbm.at[pl.ds(_eoff(i, k), B)], didx[p][k], id_sem[p]).wait()

    def _fire_gathers(p):
        for k in range(QN):
            pltpu.async_copy(x_hbm.at[sidx[p][k]], rows[p][k], gsem[p])

    def _drain_gathers(p):
        for k in range(QN):
            pltpu.make_async_copy(x_hbm.at[sidx[p][k]], rows[p][k], gsem[p]).wait()

    def _fire_scatters(p):
        for k in range(QN):
            pltpu.async_copy(rows[p][k], acc.at[didx[p][k]], ssem[p], add=True)

    def _drain_scatters(p):
        for k in range(QN):
            pltpu.make_async_copy(rows[p][k], acc.at[didx[p][k]], ssem[p]).wait()

    _fire_sidx(0, 0)
    _fire_didx(0, 0)
    _fire_sidx(1, 1)

    roff = pl.multiple_of(sid * ROWS_PT, 8)
    @pl.when(cid == 0)
    def _():
        pltpu.sync_copy(x_hbm.at[pl.ds(roff, ROWS_PT)], acc.at[pl.ds(roff, ROWS_PT)])

    @pl.when(cid == 1)
    def _():
        pltpu.sync_copy(zeros_hbm.at[pl.ds(roff, ROWS_PT)], acc.at[pl.ds(roff, ROWS_PT)])

    plsc.subcore_barrier()

    _drain_sidx(0, 0)
    _fire_gathers(0)

    def _quad_body(i, p):
        q = 1 - p

        @pl.when(i + 1 < NQUAD)
        def _():
            @pl.when(i >= 1)
            def _():
                _drain_scatters(q)

            _fire_didx(i + 1, q)
            _drain_sidx(i + 1, q)
            _fire_gathers(q)

        _drain_gathers(p)

        @pl.when(i + 2 < NQUAD)
        def _():
            _fire_sidx(i + 2, p)

        _drain_didx(i, p)
        _fire_scatters(p)

    def _quad(i, _):
        @pl.when(i % 2 == 0)
        def _():
            _quad_body(i, 0)

        @pl.when(i % 2 == 1)
        def _():
            _quad_body(i, 1)

        return 0

    lax.fori_loop(0, NQUAD, _quad, 0)

    _drain_scatters((NQUAD - 2) % 2)
    _drain_scatters((NQUAD - 1) % 2)

    eoff = pl.multiple_of(ebase + (NB - 1) * B, 8)
    pltpu.sync_copy(src_hbm.at[pl.ds(eoff, B)], sidx[0][0])
    pltpu.sync_copy(dst_hbm.at[pl.ds(eoff, B)], didx[0][0])
    pltpu.async_copy(x_hbm.at[sidx[0][0]], rows[0][0], gsem[0]).wait()
    pltpu.sync_copy(rows[0][0], acc.at[didx[0][0]], add=True)

    plsc.subcore_barrier()

    @pl.when(cid == 0)
    def _():
        pltpu.sync_copy(acc.at[pl.ds(roff, ROWS_PT)], out0.at[pl.ds(roff, ROWS_PT)])

    @pl.when(cid == 1)
    def _():
        pltpu.sync_copy(acc.at[pl.ds(roff, ROWS_PT)], out1.at[pl.ds(roff, ROWS_PT)])


def _scale_body(x_ref, d0_ref, d1_ref, o_ref):
    dis = lax.rsqrt(d0_ref[...] + d1_ref[...])
    o_ref[...] = x_ref[...] * dis


def _scale_x(x, d0, d1):
    return pl.pallas_call(
        _scale_body,
        out_shape=jax.ShapeDtypeStruct((NPAD, D), jnp.float32),
    )(x, d0, d1)


_RB = 2048


def _mid_body(r0_ref, r1_ref, d0_ref, d1_ref, w1_ref, b1_ref, w2_ref, o_ref):
    dis = lax.rsqrt(d0_ref[...] + d1_ref[...])
    y = (r0_ref[...] + r1_ref[...]) * dis
    h = jnp.dot(y, w1_ref[...], preferred_element_type=jnp.float32) + b1_ref[...]
    t = jnp.dot(h, w2_ref[...], preferred_element_type=jnp.float32)
    o_ref[...] = t * dis


def _mid(r0, r1, d0, d1, W1, b1, W2):
    grid = (NPAD // _RB,)
    row_spec = pl.BlockSpec((_RB, D), lambda i: (i, 0))
    deg_spec = pl.BlockSpec((_RB, 1), lambda i: (i, 0))
    return pl.pallas_call(
        _mid_body,
        grid=grid,
        in_specs=[
            row_spec,
            row_spec,
            deg_spec,
            deg_spec,
            pl.BlockSpec((D, 256), lambda i: (0, 0)),
            pl.BlockSpec((256,), lambda i: (0,)),
            pl.BlockSpec((256, D), lambda i: (0, 0)),
        ],
        out_specs=row_spec,
        out_shape=jax.ShapeDtypeStruct((NPAD, D), jnp.float32),
    )(r0, r1, d0, d1, W1, b1, W2)


def _final_body(q0_ref, q1_ref, d0_ref, d1_ref, b2_ref, o_ref):
    dis = lax.rsqrt(d0_ref[...] + d1_ref[...])
    o_ref[...] = jnp.maximum((q0_ref[...] + q1_ref[...]) * dis + b2_ref[...], 0.0)


def _final(q0, q1, d0, d1, b2):
    grid = (NPAD // _RB,)
    row_spec = pl.BlockSpec((_RB, D), lambda i: (i, 0))
    deg_spec = pl.BlockSpec((_RB, 1), lambda i: (i, 0))
    return pl.pallas_call(
        _final_body,
        grid=grid,
        in_specs=[
            row_spec,
            row_spec,
            deg_spec,
            deg_spec,
            pl.BlockSpec((D,), lambda i: (0,)),
        ],
        out_specs=row_spec,
        out_shape=jax.ShapeDtypeStruct((NPAD, D), jnp.float32),
    )(q0, q1, d0, d1, b2)


def kernel(X, edge_index, W1, b1, W2, b2):
    src = edge_index[0].astype(jnp.int32)
    dst = edge_index[1].astype(jnp.int32)
    xpad = jnp.pad(X, ((0, NPAD - N), (0, 0)))
    zeros = jnp.zeros((NPAD, D), jnp.float32)

    dp0, dp1 = _deg_kernel(dst)
    d0 = dp0.reshape(NPAD, 1)
    d1 = dp1.reshape(NPAD, 1)

    x0 = _scale_x(xpad, d0, d1)
    r0, r1 = _scatter_kernel(x0, zeros, src, dst)
    t0 = _mid(r0, r1, d0, d1, W1, b1, W2)
    q0, q1 = _scatter_kernel(t0, zeros, src, dst)
    return _final(q0, q1, d0, d1, b2)[:N]


# speedup vs baseline: 37.2222x; 1.0008x over previous
import functools

import jax
import jax.numpy as jnp
from jax import lax
from jax.experimental import pallas as pl
from jax.experimental.pallas import tpu as pltpu
from jax.experimental.pallas import tpu_sc as plsc

N = 10000
NPAD = 10240
E = 320000
D = 128
NC = 2
NS = 16
NW = NC * NS
EPW = E // NW
B = 80
NB = EPW // B
ROWS_PT = NPAD // NS
DROWS_PT = NPAD // NS

_mesh = plsc.VectorSubcoreMesh(core_axis_name="c", subcore_axis_name="s")


NQ = 2
NQI = NB // NQ
QN = 2
NQUAD = NB // QN


def _stage_idx(dst_buf, src_buf, off):
    for k in range(B // 16):
        o = pl.multiple_of(off + 16 * k, 8)
        dst_buf[pl.ds(pl.multiple_of(16 * k, 8), 16)] = src_buf[pl.ds(o, 16)]


@functools.partial(
    pl.kernel,
    mesh=_mesh,
    out_type=[
        jax.ShapeDtypeStruct((NPAD,), jnp.float32),
        jax.ShapeDtypeStruct((NPAD,), jnp.float32),
    ],
    scratch_types=[
        pltpu.VMEM_SHARED((NPAD,), jnp.float32),
        pltpu.VMEM((DROWS_PT,), jnp.float32),
        pltpu.VMEM((B,), jnp.float32),
        pltpu.VMEM((EPW,), jnp.int32),
    ]
    + [pltpu.VMEM((B,), jnp.int32) for _ in range(NQ)]
    + [pltpu.SemaphoreType.DMA for _ in range(NQ)],
)
def _deg_kernel(dst_hbm, out0, out1, acc, initv, val, didx_all, *rest):
    didx_b = rest[:NQ]
    ssem = rest[NQ:]
    cid = lax.axis_index("c")
    sid = lax.axis_index("s")
    wid = sid * NC + cid

    fill = jnp.where(cid == 0, 1.0, 0.0).astype(jnp.float32)
    filled = jnp.broadcast_to(fill, (16,))

    def _fill_init(i, _):
        initv[pl.ds(pl.multiple_of(i * 16, 16), 16)] = filled
        return 0

    lax.fori_loop(0, DROWS_PT // 16, _fill_init, 0)

    def _fill_ones(i, _):
        val[pl.ds(pl.multiple_of(i * 16, 16), 16)] = jnp.ones((16,), jnp.float32)
        return 0

    lax.fori_loop(0, B // 16, _fill_ones, 0)

    ebase = pl.multiple_of(wid * EPW, 8)
    pltpu.sync_copy(dst_hbm.at[pl.ds(ebase, EPW)], didx_all)

    ioff = pl.multiple_of(sid * DROWS_PT, 8)
    pltpu.sync_copy(initv, acc.at[pl.ds(ioff, DROWS_PT)])
    plsc.subcore_barrier()

    def _quad(i, _):
        for k in range(NQ):
            _stage_idx(didx_b[k], didx_all, (i * NQ + k) * B)
        descs = [
            pltpu.async_copy(val, acc.at[didx_b[k]], ssem[k], add=True)
            for k in range(NQ)
        ]
        for d in descs:
            d.wait()
        return 0

    lax.fori_loop(0, NQI, _quad, 0)
    _stage_idx(didx_b[0], didx_all, (NB - 1) * B)
    pltpu.sync_copy(val, acc.at[didx_b[0]], add=True)

    plsc.subcore_barrier()

    @pl.when(cid == 0)
    def _():
        pltpu.sync_copy(acc.at[pl.ds(ioff, DROWS_PT)], out0.at[pl.ds(ioff, DROWS_PT)])

    @pl.when(cid == 1)
    def _():
        pltpu.sync_copy(acc.at[pl.ds(ioff, DROWS_PT)], out1.at[pl.ds(ioff, DROWS_PT)])


@functools.partial(
    pl.kernel,
    mesh=_mesh,
    out_type=[
        jax.ShapeDtypeStruct((NPAD, D), jnp.float32),
        jax.ShapeDtypeStruct((NPAD, D), jnp.float32),
    ],
    scratch_types=[
        pltpu.VMEM_SHARED((NPAD, D), jnp.float32),
    ]
    + [pltpu.VMEM((B, D), jnp.float32) for _ in range(2 * QN)]
    + [pltpu.VMEM((B,), jnp.int32) for _ in range(2 * QN)]
    + [pltpu.VMEM((B,), jnp.int32) for _ in range(2 * QN)]
    + [pltpu.SemaphoreType.DMA for _ in range(8)],
)
def _scatter_kernel(x_hbm, zeros_hbm, src_hbm, dst_hbm, out0, out1,
                    acc, *rest):
    rows = [rest[QN * p:QN * p + QN] for p in range(2)]
    sidx = [rest[2 * QN + QN * p:3 * QN + QN * p] for p in range(2)]
    didx = [rest[4 * QN + QN * p:5 * QN + QN * p] for p in range(2)]
    is_sem = rest[6 * QN:6 * QN + 2]
    id_sem = rest[6 * QN + 2:6 * QN + 4]
    gsem = rest[6 * QN + 4:6 * QN + 6]
    ssem = rest[6 * QN + 6:6 * QN + 8]
    cid = lax.axis_index("c")
    sid = lax.axis_index("s")
    wid = sid * NC + cid

    ebase = pl.multiple_of(wid * EPW, 8)

    def _eoff(i, k):
        return pl.multiple_of(ebase + (i * QN + k) * B, 8)

    def _fire_sidx(i, p):
        for k in range(QN):
            pltpu.async_copy(src_hbm.at[pl.ds(_eoff(i, k), B)], sidx[p][k], is_sem[p])

    def _drain_sidx(i, p):
        for k in range(QN):
            pltpu.make_async_copy(
                src_hbm.at[pl.ds(_eoff(i, k), B)], sidx[p][k], is_sem[p]).wait()

    def _fire_didx(i, p):
        for k in range(QN):
            pltpu.async_copy(dst_hbm.at[pl.ds(_eoff(i, k), B)], didx[p][k], id_sem[p])

    def _drain_didx(i, p):
        for k in range(QN):
            pltpu.make_async_copy(
                dst_hbm.at[pl.ds(_eoff(i, k), B)], didx[p][k], id_sem[p]).wait()

    def _fire_gathers(p):
        for k in range(QN):
            pltpu.async_copy(x_hbm.at[sidx[p][k]], rows[p][k], gsem[p])

    def _drain_gathers(p):
        for k in range(QN):
            pltpu.make_async_copy(x_hbm.at[sidx[p][k]], rows[p][k], gsem[p]).wait()

    def _fire_scatters(p):
        for k in range(QN):
            pltpu.async_copy(rows[p][k], acc.at[didx[p][k]], ssem[p], add=True)

    def _drain_scatters(p):
        for k in range(QN):
            pltpu.make_async_copy(rows[p][k], acc.at[didx[p][k]], ssem[p]).wait()

    _fire_sidx(0, 0)
    _fire_didx(0, 0)
    _fire_sidx(1, 1)

    roff = pl.multiple_of(sid * ROWS_PT, 8)
    @pl.when(cid == 0)
    def _():
        pltpu.sync_copy(x_hbm.at[pl.ds(roff, ROWS_PT)], acc.at[pl.ds(roff, ROWS_PT)])

    @pl.when(cid == 1)
    def _():
        pltpu.sync_copy(zeros_hbm.at[pl.ds(roff, ROWS_PT)], acc.at[pl.ds(roff, ROWS_PT)])

    plsc.subcore_barrier()

    _drain_sidx(0, 0)
    _fire_gathers(0)

    def _quad_body(i, p):
        q = 1 - p

        @pl.when(i + 1 < NQUAD)
        def _():
            @pl.when(i >= 1)
            def _():
                _drain_scatters(q)

            _fire_didx(i + 1, q)
            _drain_sidx(i + 1, q)
            _fire_gathers(q)

        _drain_gathers(p)

        @pl.when(i + 2 < NQUAD)
        def _():
            _fire_sidx(i + 2, p)

        _drain_didx(i, p)
        _fire_scatters(p)

    def _quad(i, _):
        @pl.when(i % 2 == 0)
        def _():
            _quad_body(i, 0)

        @pl.when(i % 2 == 1)
        def _():
            _quad_body(i, 1)

        return 0

    lax.fori_loop(0, NQUAD, _quad, 0)

    _drain_scatters((NQUAD - 2) % 2)
    _drain_scatters((NQUAD - 1) % 2)

    eoff = pl.multiple_of(ebase + (NB - 1) * B, 8)
    pltpu.sync_copy(src_hbm.at[pl.ds(eoff, B)], sidx[0][0])
    pltpu.sync_copy(dst_hbm.at[pl.ds(eoff, B)], didx[0][0])
    pltpu.async_copy(x_hbm.at[sidx[0][0]], rows[0][0], gsem[0]).wait()
    pltpu.sync_copy(rows[0][0], acc.at[didx[0][0]], add=True)

    plsc.subcore_barrier()

    @pl.when(cid == 0)
    def _():
        pltpu.sync_copy(acc.at[pl.ds(roff, ROWS_PT)], out0.at[pl.ds(roff, ROWS_PT)])

    @pl.when(cid == 1)
    def _():
        pltpu.sync_copy(acc.at[pl.ds(roff, ROWS_PT)], out1.at[pl.ds(roff, ROWS_PT)])


def _scale_body(x_ref, d0_ref, d1_ref, o_ref):
    dis = lax.rsqrt(d0_ref[...] + d1_ref[...])
    o_ref[...] = x_ref[...] * dis


def _scale_x(x, d0, d1):
    return pl.pallas_call(
        _scale_body,
        out_shape=jax.ShapeDtypeStruct((NPAD, D), jnp.float32),
    )(x, d0, d1)


_RB = 2048


def _mid_body(r0_ref, r1_ref, d0_ref, d1_ref, w1_ref, b1_ref, w2_ref, o_ref):
    dis = lax.rsqrt(d0_ref[...] + d1_ref[...])
    y = (r0_ref[...] + r1_ref[...]) * dis
    m = jnp.dot(w1_ref[...], w2_ref[...], preferred_element_type=jnp.float32)
    t = jnp.dot(y, m, preferred_element_type=jnp.float32)
    t = t + jnp.dot(b1_ref[...].reshape(1, 256), w2_ref[...],
                    preferred_element_type=jnp.float32)
    o_ref[...] = t * dis


def _mid(r0, r1, d0, d1, W1, b1, W2):
    grid = (NPAD // _RB,)
    row_spec = pl.BlockSpec((_RB, D), lambda i: (i, 0))
    deg_spec = pl.BlockSpec((_RB, 1), lambda i: (i, 0))
    return pl.pallas_call(
        _mid_body,
        grid=grid,
        in_specs=[
            row_spec,
            row_spec,
            deg_spec,
            deg_spec,
            pl.BlockSpec((D, 256), lambda i: (0, 0)),
            pl.BlockSpec((256,), lambda i: (0,)),
            pl.BlockSpec((256, D), lambda i: (0, 0)),
        ],
        out_specs=row_spec,
        out_shape=jax.ShapeDtypeStruct((NPAD, D), jnp.float32),
    )(r0, r1, d0, d1, W1, b1, W2)


def _final_body(q0_ref, q1_ref, d0_ref, d1_ref, b2_ref, o_ref):
    dis = lax.rsqrt(d0_ref[...] + d1_ref[...])
    o_ref[...] = jnp.maximum((q0_ref[...] + q1_ref[...]) * dis + b2_ref[...], 0.0)


def _final(q0, q1, d0, d1, b2):
    grid = (NPAD // _RB,)
    row_spec = pl.BlockSpec((_RB, D), lambda i: (i, 0))
    deg_spec = pl.BlockSpec((_RB, 1), lambda i: (i, 0))
    return pl.pallas_call(
        _final_body,
        grid=grid,
        in_specs=[
            row_spec,
            row_spec,
            deg_spec,
            deg_spec,
            pl.BlockSpec((D,), lambda i: (0,)),
        ],
        out_specs=row_spec,
        out_shape=jax.ShapeDtypeStruct((NPAD, D), jnp.float32),
    )(q0, q1, d0, d1, b2)


def kernel(X, edge_index, W1, b1, W2, b2):
    src = edge_index[0].astype(jnp.int32)
    dst = edge_index[1].astype(jnp.int32)
    xpad = jnp.pad(X, ((0, NPAD - N), (0, 0)))
    zeros = jnp.zeros((NPAD, D), jnp.float32)

    dp0, dp1 = _deg_kernel(dst)
    d0 = dp0.reshape(NPAD, 1)
    d1 = dp1.reshape(NPAD, 1)

    x0 = _scale_x(xpad, d0, d1)
    r0, r1 = _scatter_kernel(x0, zeros, src, dst)
    t0 = _mid(r0, r1, d0, d1, W1, b1, W2)
    q0, q1 = _scatter_kernel(t0, zeros, src, dst)
    return _final(q0, q1, d0, d1, b2)[:N]
